# baseline (speedup 1.0000x reference)
import functools

import jax
import jax.numpy as jnp
from jax import lax
from jax.experimental import pallas as pl
from jax.experimental.pallas import tpu as pltpu
from jax.experimental.pallas import tpu_sc as plsc

NC = 2
NS = 16
LANES = 16
ZCH = 25


def _make_sc_agg(n_nodes, d_row, e_total, ch):
    nw = NC * NS
    ept = e_total // nw
    nch = ept // ch
    nch2 = nch // 2
    rpt = n_nodes // NS
    rq = rpt // ZCH

    mesh = plsc.VectorSubcoreMesh(core_axis_name="c", subcore_axis_name="s")

    @functools.partial(
        pl.kernel,
        out_type=jax.ShapeDtypeStruct((NC * n_nodes, d_row), jnp.float32),
        mesh=mesh,
        scratch_types=[
            pltpu.VMEM((nch2, ch), jnp.int32),
            pltpu.VMEM((nch2, ch), jnp.int32),
            pltpu.VMEM((ch, d_row), jnp.float32),
            pltpu.VMEM((ch, d_row), jnp.float32),
            pltpu.VMEM_SHARED((n_nodes, d_row), jnp.float32),
            pltpu.SemaphoreType.DMA,
            pltpu.SemaphoreType.DMA,
        ],
        compiler_params=pltpu.CompilerParams(use_tc_tiling_on_sc=False),
    )
    def sc_agg(feat_hbm, src_hbm, dst_hbm, out_hbm,
               idx_s, idx_d, rows0, rows1, agg, gs0, gs1):
        c = lax.axis_index("c")
        s = lax.axis_index("s")
        w = c * NS + s

        def zero_row(r, carry):
            for k in range(d_row // LANES):
                rows0[r, pl.ds(k * LANES, LANES)] = jnp.zeros(
                    (LANES,), jnp.float32)
            return carry
        lax.fori_loop(0, ZCH, zero_row, 0)
        zrow = rows0.at[pl.ds(0, ZCH)]
        for q in range(rq):
            pltpu.sync_copy(zrow, agg.at[pl.ds(s * rpt + q * ZCH, ZCH)])
        plsc.subcore_barrier()

        for half in range(2):
            pltpu.sync_copy(
                src_hbm.at[pl.ds(w * nch + half * nch2, nch2)], idx_s)
            pltpu.sync_copy(
                dst_hbm.at[pl.ds(w * nch + half * nch2, nch2)], idx_d)

            pltpu.async_copy(feat_hbm.at[idx_s.at[0]], rows0, gs0)

            def step(t, carry):
                j0 = 2 * t
                pltpu.async_copy(feat_hbm.at[idx_s.at[j0 + 1]], rows1, gs1)
                pltpu.make_async_copy(
                    feat_hbm.at[idx_s.at[j0]], rows0, gs0).wait()
                pltpu.sync_copy(rows0, agg.at[idx_d.at[j0]], add=True)

                @pl.when(t < nch2 // 2 - 1)
                def _():
                    pltpu.async_copy(
                        feat_hbm.at[idx_s.at[j0 + 2]], rows0, gs0)

                pltpu.make_async_copy(
                    feat_hbm.at[idx_s.at[j0 + 1]], rows1, gs1).wait()
                pltpu.sync_copy(rows1, agg.at[idx_d.at[j0 + 1]], add=True)
                return carry

            lax.fori_loop(0, nch2 // 2, step, 0)

        plsc.subcore_barrier()

        pltpu.sync_copy(
            agg.at[pl.ds(s * rpt, rpt)],
            out_hbm.at[pl.ds(c * n_nodes + s * rpt, rpt)])

    return sc_agg


def _matmul_bias(a, W, b):
    n, d = a.shape
    bm = 400
    grid = n // bm

    def body(ar, w, br, y_ref):
        y_ref[...] = lax.dot_general(
            ar[...], w[...], (((1,), (1,)), ((), ())),
            preferred_element_type=jnp.float32) + br[...]

    return pl.pallas_call(
        body,
        grid=(grid,),
        in_specs=[
            pl.BlockSpec((bm, d), lambda i: (i, 0)),
            pl.BlockSpec((d, d), lambda i: (0, 0)),
            pl.BlockSpec((1, d), lambda i: (0, 0)),
        ],
        out_specs=pl.BlockSpec((bm, d), lambda i: (i, 0)),
        out_shape=jax.ShapeDtypeStruct((n, d), jnp.float32),
    )(a, W, b)


def _combine1(p1, xr, Wl):
    n, d = xr.shape
    dr = p1.shape[1]
    bm = 400
    grid = n // bm

    def body(pa, pb, xrr, wl, h_ref, inv_ref):
        agg = pa[:, :d] + pb[:, :d]
        cnt = pa[:, d:d + 1] + pb[:, d:d + 1]
        inv = 1.0 / jnp.maximum(cnt, 1.0)
        mean = agg * inv
        mm = lax.dot_general(mean, wl[...], (((1,), (1,)), ((), ())),
                             preferred_element_type=jnp.float32)
        h_ref[...] = jnp.maximum(mm + xrr[...], 0.0)
        inv_ref[...] = jnp.broadcast_to(inv, (bm, 8))

    return pl.pallas_call(
        body,
        grid=(grid,),
        in_specs=[
            pl.BlockSpec((bm, dr), lambda i: (i, 0)),
            pl.BlockSpec((bm, dr), lambda i, g=grid: (i + g, 0)),
            pl.BlockSpec((bm, d), lambda i: (i, 0)),
            pl.BlockSpec((d, d), lambda i: (0, 0)),
        ],
        out_specs=[
            pl.BlockSpec((bm, d), lambda i: (i, 0)),
            pl.BlockSpec((bm, 8), lambda i: (i, 0)),
        ],
        out_shape=[
            jax.ShapeDtypeStruct((n, d), jnp.float32),
            jax.ShapeDtypeStruct((n, 8), jnp.float32),
        ],
    )(p1, p1, xr, Wl)


def _combine2(p2, hr, inv8, Wl):
    n, d = hr.shape
    bm = 400
    grid = n // bm

    def body(pa, pb, hrr, invr, wl, out_ref):
        mean = (pa[...] + pb[...]) * invr[:, 0:1]
        mm = lax.dot_general(mean, wl[...], (((1,), (1,)), ((), ())),
                             preferred_element_type=jnp.float32)
        out_ref[...] = mm + hrr[...]

    return pl.pallas_call(
        body,
        grid=(grid,),
        in_specs=[
            pl.BlockSpec((bm, d), lambda i: (i, 0)),
            pl.BlockSpec((bm, d), lambda i, g=grid: (i + g, 0)),
            pl.BlockSpec((bm, d), lambda i: (i, 0)),
            pl.BlockSpec((bm, 8), lambda i: (i, 0)),
            pl.BlockSpec((d, d), lambda i: (0, 0)),
        ],
        out_specs=pl.BlockSpec((bm, d), lambda i: (i, 0)),
        out_shape=jax.ShapeDtypeStruct((n, d), jnp.float32),
    )(p2, p2, hr, inv8, Wl)


def kernel(x, edge_index, W1l, b1l, W1r, W2l, b2l, W2r):
    n, d = x.shape
    e = edge_index.shape[1]
    d_aug = d + LANES
    ch1, ch2 = 100, 125

    src1 = edge_index[0].reshape(e // ch1, ch1)
    dst1 = edge_index[1].reshape(e // ch1, ch1)
    src2 = edge_index[0].reshape(e // ch2, ch2)
    dst2 = edge_index[1].reshape(e // ch2, ch2)

    x_aug = jnp.concatenate(
        [x, jnp.ones((n, 1), jnp.float32),
         jnp.zeros((n, LANES - 1), jnp.float32)], axis=1)

    p1 = _make_sc_agg(n, d_aug, e, ch1)(x_aug, src1, dst1)
    xr = _matmul_bias(x, W1r, b1l.reshape(1, d))
    h, inv8 = _combine1(p1, xr, W1l)
    p2 = _make_sc_agg(n, d, e, ch2)(h, src2, dst2)
    hr = _matmul_bias(h, W2r, b2l.reshape(1, d))
    out = _combine2(p2, hr, inv8, W2l)
    return out


# speedup vs baseline: 1.0124x; 1.0124x over previous
import functools

import jax
import jax.numpy as jnp
from jax import lax
from jax.experimental import pallas as pl
from jax.experimental.pallas import tpu as pltpu
from jax.experimental.pallas import tpu_sc as plsc

NC = 2
NS = 16
LANES = 16
ZCH = 25


def _make_sc_agg(n_nodes, d_row, e_total, ch):
    nw = NC * NS
    ept = e_total // nw
    nch = ept // ch
    nch2 = nch // 2
    rpt = n_nodes // NS
    rq = rpt // ZCH

    mesh = plsc.VectorSubcoreMesh(core_axis_name="c", subcore_axis_name="s")

    @functools.partial(
        pl.kernel,
        out_type=jax.ShapeDtypeStruct((NC * n_nodes, d_row), jnp.float32),
        mesh=mesh,
        scratch_types=[
            pltpu.VMEM((nch2, ch), jnp.int32),
            pltpu.VMEM((nch2, ch), jnp.int32),
            pltpu.VMEM((ch, d_row), jnp.float32),
            pltpu.VMEM((ch, d_row), jnp.float32),
            pltpu.VMEM_SHARED((n_nodes, d_row), jnp.float32),
            pltpu.SemaphoreType.DMA,
            pltpu.SemaphoreType.DMA,
        ],
        compiler_params=pltpu.CompilerParams(use_tc_tiling_on_sc=False),
    )
    def sc_agg(feat_hbm, src_hbm, dst_hbm, out_hbm,
               idx_s, idx_d, rows0, rows1, agg, gs0, gs1):
        c = lax.axis_index("c")
        s = lax.axis_index("s")
        w = c * NS + s

        def zero_row(r, carry):
            for k in range(d_row // LANES):
                rows0[r, pl.ds(k * LANES, LANES)] = jnp.zeros(
                    (LANES,), jnp.float32)
            return carry
        lax.fori_loop(0, ch, zero_row, 0)
        nfull = rpt // ch
        tail = rpt % ch
        zcopies = []
        for q in range(nfull):
            zcopies.append((rows0, agg.at[pl.ds(s * rpt + q * ch, ch)]))
        if tail:
            zcopies.append((rows0.at[pl.ds(0, tail)],
                            agg.at[pl.ds(s * rpt + nfull * ch, tail)]))
        for zsrc, zdst in zcopies:
            pltpu.async_copy(zsrc, zdst, gs0)
        for zsrc, zdst in zcopies:
            pltpu.make_async_copy(zsrc, zdst, gs0).wait()
        plsc.subcore_barrier()

        for half in range(2):
            src_half = src_hbm.at[pl.ds(w * nch + half * nch2, nch2)]
            dst_half = dst_hbm.at[pl.ds(w * nch + half * nch2, nch2)]
            pltpu.async_copy(src_half, idx_s, gs0)
            pltpu.async_copy(dst_half, idx_d, gs1)
            pltpu.make_async_copy(src_half, idx_s, gs0).wait()
            pltpu.make_async_copy(dst_half, idx_d, gs1).wait()

            pltpu.async_copy(feat_hbm.at[idx_s.at[0]], rows0, gs0)

            def step(t, carry):
                j0 = 2 * t
                pltpu.async_copy(feat_hbm.at[idx_s.at[j0 + 1]], rows1, gs1)
                pltpu.make_async_copy(
                    feat_hbm.at[idx_s.at[j0]], rows0, gs0).wait()
                pltpu.sync_copy(rows0, agg.at[idx_d.at[j0]], add=True)

                @pl.when(t < nch2 // 2 - 1)
                def _():
                    pltpu.async_copy(
                        feat_hbm.at[idx_s.at[j0 + 2]], rows0, gs0)

                pltpu.make_async_copy(
                    feat_hbm.at[idx_s.at[j0 + 1]], rows1, gs1).wait()
                pltpu.sync_copy(rows1, agg.at[idx_d.at[j0 + 1]], add=True)
                return carry

            lax.fori_loop(0, nch2 // 2, step, 0)

        plsc.subcore_barrier()

        pltpu.sync_copy(
            agg.at[pl.ds(s * rpt, rpt)],
            out_hbm.at[pl.ds(c * n_nodes + s * rpt, rpt)])

    return sc_agg


def _matmul_bias(a, W, b):
    n, d = a.shape
    bm = 400
    grid = n // bm

    def body(ar, w, br, y_ref):
        y_ref[...] = lax.dot_general(
            ar[...], w[...], (((1,), (1,)), ((), ())),
            preferred_element_type=jnp.float32) + br[...]

    return pl.pallas_call(
        body,
        grid=(grid,),
        in_specs=[
            pl.BlockSpec((bm, d), lambda i: (i, 0)),
            pl.BlockSpec((d, d), lambda i: (0, 0)),
            pl.BlockSpec((1, d), lambda i: (0, 0)),
        ],
        out_specs=pl.BlockSpec((bm, d), lambda i: (i, 0)),
        out_shape=jax.ShapeDtypeStruct((n, d), jnp.float32),
    )(a, W, b)


def _combine1(p1, xr, Wl):
    n, d = xr.shape
    dr = p1.shape[1]
    bm = 400
    grid = n // bm

    def body(pa, pb, xrr, wl, h_ref, inv_ref):
        agg = pa[:, :d] + pb[:, :d]
        cnt = pa[:, d:d + 1] + pb[:, d:d + 1]
        inv = 1.0 / jnp.maximum(cnt, 1.0)
        mean = agg * inv
        mm = lax.dot_general(mean, wl[...], (((1,), (1,)), ((), ())),
                             preferred_element_type=jnp.float32)
        h_ref[...] = jnp.maximum(mm + xrr[...], 0.0)
        inv_ref[...] = jnp.broadcast_to(inv, (bm, 8))

    return pl.pallas_call(
        body,
        grid=(grid,),
        in_specs=[
            pl.BlockSpec((bm, dr), lambda i: (i, 0)),
            pl.BlockSpec((bm, dr), lambda i, g=grid: (i + g, 0)),
            pl.BlockSpec((bm, d), lambda i: (i, 0)),
            pl.BlockSpec((d, d), lambda i: (0, 0)),
        ],
        out_specs=[
            pl.BlockSpec((bm, d), lambda i: (i, 0)),
            pl.BlockSpec((bm, 8), lambda i: (i, 0)),
        ],
        out_shape=[
            jax.ShapeDtypeStruct((n, d), jnp.float32),
            jax.ShapeDtypeStruct((n, 8), jnp.float32),
        ],
    )(p1, p1, xr, Wl)


def _combine2(p2, hr, inv8, Wl):
    n, d = hr.shape
    bm = 400
    grid = n // bm

    def body(pa, pb, hrr, invr, wl, out_ref):
        mean = (pa[...] + pb[...]) * invr[:, 0:1]
        mm = lax.dot_general(mean, wl[...], (((1,), (1,)), ((), ())),
                             preferred_element_type=jnp.float32)
        out_ref[...] = mm + hrr[...]

    return pl.pallas_call(
        body,
        grid=(grid,),
        in_specs=[
            pl.BlockSpec((bm, d), lambda i: (i, 0)),
            pl.BlockSpec((bm, d), lambda i, g=grid: (i + g, 0)),
            pl.BlockSpec((bm, d), lambda i: (i, 0)),
            pl.BlockSpec((bm, 8), lambda i: (i, 0)),
            pl.BlockSpec((d, d), lambda i: (0, 0)),
        ],
        out_specs=pl.BlockSpec((bm, d), lambda i: (i, 0)),
        out_shape=jax.ShapeDtypeStruct((n, d), jnp.float32),
    )(p2, p2, hr, inv8, Wl)


def kernel(x, edge_index, W1l, b1l, W1r, W2l, b2l, W2r):
    n, d = x.shape
    e = edge_index.shape[1]
    d_aug = d + LANES
    ch1, ch2 = 100, 125

    src1 = edge_index[0].reshape(e // ch1, ch1)
    dst1 = edge_index[1].reshape(e // ch1, ch1)
    src2 = edge_index[0].reshape(e // ch2, ch2)
    dst2 = edge_index[1].reshape(e // ch2, ch2)

    x_aug = jnp.concatenate(
        [x, jnp.ones((n, 1), jnp.float32),
         jnp.zeros((n, LANES - 1), jnp.float32)], axis=1)

    p1 = _make_sc_agg(n, d_aug, e, ch1)(x_aug, src1, dst1)
    xr = _matmul_bias(x, W1r, b1l.reshape(1, d))
    h, inv8 = _combine1(p1, xr, W1l)
    p2 = _make_sc_agg(n, d, e, ch2)(h, src2, dst2)
    hr = _matmul_bias(h, W2r, b2l.reshape(1, d))
    out = _combine2(p2, hr, inv8, W2l)
    return out


# speedup vs baseline: 1.1571x; 1.1429x over previous
import functools

import jax
import jax.numpy as jnp
from jax import lax
from jax.experimental import pallas as pl
from jax.experimental.pallas import tpu as pltpu
from jax.experimental.pallas import tpu_sc as plsc

NC = 2
NS = 16
LANES = 16
ZCH = 25


def _make_sc_agg(n_nodes, d_row, e_total, ch, counts=False):
    nw = NC * NS
    ept = e_total // nw
    nch = ept // ch
    nch2 = nch // 2
    rpt = n_nodes // NS
    hr = n_nodes // LANES
    nfv = ch // LANES
    tail = ch % LANES
    mq = hr // 125

    mesh = plsc.VectorSubcoreMesh(core_axis_name="c", subcore_axis_name="s")

    out_types = [jax.ShapeDtypeStruct((NC * n_nodes, d_row), jnp.float32)]
    extra_scratch = []
    if counts:
        out_types.append(
            jax.ShapeDtypeStruct((NC * hr, LANES), jnp.float32))
        extra_scratch = [
            pltpu.VMEM((hr, LANES), jnp.float32),
            pltpu.VMEM((mq, 125), jnp.int32),
            pltpu.VMEM_SHARED((hr, LANES), jnp.float32),
        ]

    @functools.partial(
        pl.kernel,
        out_type=out_types if counts else out_types[0],
        mesh=mesh,
        scratch_types=[
            pltpu.VMEM((nch2, ch), jnp.int32),
            pltpu.VMEM((nch2, ch), jnp.int32),
            pltpu.VMEM((ch, d_row), jnp.float32),
            pltpu.VMEM((ch, d_row), jnp.float32),
            pltpu.VMEM_SHARED((n_nodes, d_row), jnp.float32),
            pltpu.SemaphoreType.DMA,
            pltpu.SemaphoreType.DMA,
        ] + extra_scratch,
        compiler_params=pltpu.CompilerParams(
            use_tc_tiling_on_sc=False,
            needs_layout_passes=not counts),
    )
    def sc_agg(feat_hbm, src_hbm, dst_hbm, *outs_and_scratch):
        if counts:
            (out_hbm, cnt_hbm, idx_s, idx_d, rows0, rows1, agg, gs0, gs1,
             hist, ridx, cnt_sp) = outs_and_scratch
        else:
            (out_hbm, idx_s, idx_d, rows0, rows1, agg,
             gs0, gs1) = outs_and_scratch
        c = lax.axis_index("c")
        s = lax.axis_index("s")
        w = c * NS + s

        ones16 = jnp.full((LANES,), 1.0, jnp.float32)
        lane = lax.iota(jnp.int32, LANES)
        tailmask = lane >= (LANES - tail)

        def hist_row(j):
            for k in range(nfv):
                v = idx_d[j, pl.ds(k * LANES, LANES)]
                plsc.addupdate_scatter(
                    hist, [v >> 4, v & (LANES - 1)], ones16)
            if tail:
                v = idx_d[j, pl.ds(ch - LANES, LANES)]
                plsc.addupdate_scatter(
                    hist, [v >> 4, v & (LANES - 1)], ones16, mask=tailmask)

        def zero_row(r, carry):
            for k in range(d_row // LANES):
                rows0[r, pl.ds(k * LANES, LANES)] = jnp.zeros(
                    (LANES,), jnp.float32)
            return carry
        lax.fori_loop(0, ch, zero_row, 0)
        nfull = rpt // ch
        ztail = rpt % ch
        zcopies = []
        for q in range(nfull):
            zcopies.append((rows0, agg.at[pl.ds(s * rpt + q * ch, ch)]))
        if ztail:
            zcopies.append((rows0.at[pl.ds(0, ztail)],
                            agg.at[pl.ds(s * rpt + nfull * ch, ztail)]))
        for zsrc, zdst in zcopies:
            pltpu.async_copy(zsrc, zdst, gs0)
        if counts:
            def zero_hist(r, carry):
                hist[r, pl.ds(0, LANES)] = jnp.zeros((LANES,), jnp.float32)
                return carry
            lax.fori_loop(0, hr, zero_hist, 0)
            for q in range(mq):
                rows_here = min(125, hr - q * 125)
                for k in range(0, rows_here - LANES + 1, LANES):
                    ridx[q, pl.ds(k, LANES)] = lane + (q * 125 + k)
                if rows_here % LANES:
                    off = rows_here - LANES
                    ridx[q, pl.ds(off, LANES)] = lane + (q * 125 + off)

            @pl.when(s == 0)
            def _():
                pltpu.sync_copy(hist, cnt_sp)
        for zsrc, zdst in zcopies:
            pltpu.make_async_copy(zsrc, zdst, gs0).wait()
        plsc.subcore_barrier()

        for half in range(2):
            src_half = src_hbm.at[pl.ds(w * nch + half * nch2, nch2)]
            dst_half = dst_hbm.at[pl.ds(w * nch + half * nch2, nch2)]
            pltpu.async_copy(src_half, idx_s, gs0)
            pltpu.async_copy(dst_half, idx_d, gs1)
            pltpu.make_async_copy(src_half, idx_s, gs0).wait()
            pltpu.make_async_copy(dst_half, idx_d, gs1).wait()

            pltpu.async_copy(feat_hbm.at[idx_s.at[0]], rows0, gs0)

            def step(t, carry):
                j0 = 2 * t
                pltpu.async_copy(feat_hbm.at[idx_s.at[j0 + 1]], rows1, gs1)
                if counts:
                    hist_row(j0)
                pltpu.make_async_copy(
                    feat_hbm.at[idx_s.at[j0]], rows0, gs0).wait()
                pltpu.sync_copy(rows0, agg.at[idx_d.at[j0]], add=True)

                @pl.when(t < nch2 // 2 - 1)
                def _():
                    pltpu.async_copy(
                        feat_hbm.at[idx_s.at[j0 + 2]], rows0, gs0)

                if counts:
                    hist_row(j0 + 1)
                pltpu.make_async_copy(
                    feat_hbm.at[idx_s.at[j0 + 1]], rows1, gs1).wait()
                pltpu.sync_copy(rows1, agg.at[idx_d.at[j0 + 1]], add=True)
                return carry

            lax.fori_loop(0, nch2 // 2, step, 0)

        if counts:
            for q in range(mq):
                pltpu.sync_copy(
                    hist.at[pl.ds(q * 125, 125)],
                    cnt_sp.at[ridx.at[q]], add=True)
        plsc.subcore_barrier()

        pltpu.sync_copy(
            agg.at[pl.ds(s * rpt, rpt)],
            out_hbm.at[pl.ds(c * n_nodes + s * rpt, rpt)])
        if counts:
            @pl.when(s == 0)
            def _():
                pltpu.sync_copy(cnt_sp, cnt_hbm.at[pl.ds(c * hr, hr)])

    return sc_agg


def _matmul_bias(a, W, b):
    n, d = a.shape
    bm = 400
    grid = n // bm

    def body(ar, w, br, y_ref):
        y_ref[...] = lax.dot_general(
            ar[...], w[...], (((1,), (1,)), ((), ())),
            preferred_element_type=jnp.float32) + br[...]

    return pl.pallas_call(
        body,
        grid=(grid,),
        in_specs=[
            pl.BlockSpec((bm, d), lambda i: (i, 0)),
            pl.BlockSpec((d, d), lambda i: (0, 0)),
            pl.BlockSpec((1, d), lambda i: (0, 0)),
        ],
        out_specs=pl.BlockSpec((bm, d), lambda i: (i, 0)),
        out_shape=jax.ShapeDtypeStruct((n, d), jnp.float32),
    )(a, W, b)


def _combine1(p1, cnta, cntb, xr, Wl):
    n, d = xr.shape
    bm = 400
    grid = n // bm

    def body(pa, pb, ca, cb, xrr, wl, h_ref, inv_ref):
        cnt = ca[...] + cb[...]
        inv = 1.0 / jnp.maximum(cnt, 1.0)
        mean = (pa[...] + pb[...]) * inv
        mm = lax.dot_general(mean, wl[...], (((1,), (1,)), ((), ())),
                             preferred_element_type=jnp.float32)
        h_ref[...] = jnp.maximum(mm + xrr[...], 0.0)
        inv_ref[...] = jnp.broadcast_to(inv, (bm, 8))

    return pl.pallas_call(
        body,
        grid=(grid,),
        in_specs=[
            pl.BlockSpec((bm, d), lambda i: (i, 0)),
            pl.BlockSpec((bm, d), lambda i, g=grid: (i + g, 0)),
            pl.BlockSpec((bm, 1), lambda i: (i, 0)),
            pl.BlockSpec((bm, 1), lambda i: (i, 0)),
            pl.BlockSpec((bm, d), lambda i: (i, 0)),
            pl.BlockSpec((d, d), lambda i: (0, 0)),
        ],
        out_specs=[
            pl.BlockSpec((bm, d), lambda i: (i, 0)),
            pl.BlockSpec((bm, 8), lambda i: (i, 0)),
        ],
        out_shape=[
            jax.ShapeDtypeStruct((n, d), jnp.float32),
            jax.ShapeDtypeStruct((n, 8), jnp.float32),
        ],
    )(p1, p1, cnta, cntb, xr, Wl)


def _combine2(p2, hr, inv8, Wl):
    n, d = hr.shape
    bm = 400
    grid = n // bm

    def body(pa, pb, hrr, invr, wl, out_ref):
        mean = (pa[...] + pb[...]) * invr[:, 0:1]
        mm = lax.dot_general(mean, wl[...], (((1,), (1,)), ((), ())),
                             preferred_element_type=jnp.float32)
        out_ref[...] = mm + hrr[...]

    return pl.pallas_call(
        body,
        grid=(grid,),
        in_specs=[
            pl.BlockSpec((bm, d), lambda i: (i, 0)),
            pl.BlockSpec((bm, d), lambda i, g=grid: (i + g, 0)),
            pl.BlockSpec((bm, d), lambda i: (i, 0)),
            pl.BlockSpec((bm, 8), lambda i: (i, 0)),
            pl.BlockSpec((d, d), lambda i: (0, 0)),
        ],
        out_specs=pl.BlockSpec((bm, d), lambda i: (i, 0)),
        out_shape=jax.ShapeDtypeStruct((n, d), jnp.float32),
    )(p2, p2, hr, inv8, Wl)


def kernel(x, edge_index, W1l, b1l, W1r, W2l, b2l, W2r):
    n, d = x.shape
    e = edge_index.shape[1]
    ch1, ch2 = 100, 125

    src1 = edge_index[0].reshape(e // ch1, ch1)
    dst1 = edge_index[1].reshape(e // ch1, ch1)
    src2 = edge_index[0].reshape(e // ch2, ch2)
    dst2 = edge_index[1].reshape(e // ch2, ch2)

    p1, pc = _make_sc_agg(n, d, e, ch1, counts=True)(x, src1, dst1)
    pcr = pc.reshape(NC, n, 1)
    xr = _matmul_bias(x, W1r, b1l.reshape(1, d))
    h, inv8 = _combine1(p1, pcr[0], pcr[1], xr, W1l)
    p2 = _make_sc_agg(n, d, e, ch2)(h, src2, dst2)
    hr = _matmul_bias(h, W2r, b2l.reshape(1, d))
    out = _combine2(p2, hr, inv8, W2l)
    return out
